# Initial kernel scaffold; baseline (speedup 1.0000x reference)
import jax
import jax.numpy as jnp
from jax.experimental import pallas as pl


def kernel(h, x, edges, vel, edge_attr, params):
    raise NotImplementedError("write your pallas kernel here")


# speedup vs baseline: 2.1137x; 2.1137x over previous
import functools

import jax
import jax.numpy as jnp
from jax import lax
from jax.experimental import pallas as pl
from jax.experimental.pallas import tpu as pltpu
from jax.experimental.pallas import tpu_sc as plsc

N = 10000
NPAD = 10240
E = 320000
ECH = 128
NCH = E // ECH
HN = 256
CW = 1.0

_NW = 32
_ROWS_PER_TILE = NPAD // 16


def _gather_body(row_hbm, col_hbm, ha_hbm, hb_hbm, cps_hbm,
                 har_o, hbc_o, diff_o,
                 idxr, idxc, bufa, bufb, bufd, px, py, pz,
                 sema, semb):
    c = lax.axis_index("c")
    s = lax.axis_index("s")
    wid = s * 2 + c

    pltpu.sync_copy(cps_hbm.at[0], px)
    pltpu.sync_copy(cps_hbm.at[1], py)
    pltpu.sync_copy(cps_hbm.at[2], pz)
    zv = jnp.zeros((16,), jnp.float32)

    def zr(i, carry):
        bufd[i, :] = zv
        return carry

    lax.fori_loop(0, ECH, zr, 0)

    def body(k, carry):
        j = wid + k * _NW

        @pl.when(j < NCH)
        def _():
            base = j * ECH
            pltpu.sync_copy(row_hbm.at[pl.ds(base, ECH)], idxr)
            pltpu.sync_copy(col_hbm.at[pl.ds(base, ECH)], idxc)
            cpa = pltpu.async_copy(ha_hbm.at[idxr], bufa, sema)
            cpb = pltpu.async_copy(hb_hbm.at[idxc], bufb, semb)

            def grp(g, carry2):
                r16 = idxr[pl.ds(g * 16, 16)]
                c16 = idxc[pl.ds(g * 16, 16)]
                l16 = lax.iota(jnp.int32, 16) + g * 16
                for d, plane in enumerate((px, py, pz)):
                    cd = jnp.full((16,), d, jnp.int32)
                    dv = (plsc.load_gather(plane, [r16])
                          - plsc.load_gather(plane, [c16]))
                    plsc.store_scatter(bufd, [l16, cd], dv)
                return carry2

            lax.fori_loop(0, ECH // 16, grp, 0)
            cpa.wait()
            cpb.wait()
            pltpu.sync_copy(bufa, har_o.at[pl.ds(base, ECH)])
            pltpu.sync_copy(bufb, hbc_o.at[pl.ds(base, ECH)])
            pltpu.sync_copy(bufd, diff_o.at[pl.ds(base, ECH)])

        return carry

    lax.fori_loop(0, (NCH + _NW - 1) // _NW, body, 0)


@functools.cache
def _get_gather_call():
  return functools.partial(
    pl.kernel,
    out_type=[
        jax.ShapeDtypeStruct((E, HN), jnp.float32),
        jax.ShapeDtypeStruct((E, HN), jnp.float32),
        jax.ShapeDtypeStruct((E, 16), jnp.float32),
    ],
    mesh=plsc.VectorSubcoreMesh(core_axis_name="c", subcore_axis_name="s"),
    compiler_params=pltpu.CompilerParams(needs_layout_passes=False),
    scratch_types=[
        pltpu.VMEM((ECH,), jnp.int32),
        pltpu.VMEM((ECH,), jnp.int32),
        pltpu.VMEM((ECH, HN), jnp.float32),
        pltpu.VMEM((ECH, HN), jnp.float32),
        pltpu.VMEM((ECH, 16), jnp.float32),
        pltpu.VMEM((NPAD,), jnp.float32),
        pltpu.VMEM((NPAD,), jnp.float32),
        pltpu.VMEM((NPAD,), jnp.float32),
        pltpu.SemaphoreType.DMA,
        pltpu.SemaphoreType.DMA,
    ],
  )(_gather_body)


def _scatter_body(row_hbm, m_hbm, tr_hbm, zm_hbm,
                  outm, outt,
                  idxv, bufm, acc_m):
    c = lax.axis_index("c")
    s = lax.axis_index("s")
    lo = c * 128
    rs = s * _ROWS_PER_TILE
    pltpu.sync_copy(zm_hbm, bufm)

    def zcp(t, carry):
        pltpu.sync_copy(bufm, acc_m.at[pl.ds(rs + t * ECH, ECH)])
        return carry

    lax.fori_loop(0, _ROWS_PER_TILE // ECH, zcp, 0)
    plsc.subcore_barrier()

    def body(k, carry):
        j = s + k * 16

        @pl.when(j < NCH)
        def _():
            base = j * ECH
            pltpu.sync_copy(row_hbm.at[pl.ds(base, ECH)], idxv)
            pltpu.sync_copy(m_hbm.at[pl.ds(base, ECH), pl.ds(lo, 128)], bufm)
            pltpu.sync_copy(bufm, acc_m.at[idxv], add=True)

        return carry

    lax.fori_loop(0, (NCH + 15) // 16, body, 0)
    plsc.subcore_barrier()

    def drain(t, carry):
        ro = rs + t * ECH
        pltpu.sync_copy(acc_m.at[pl.ds(ro, ECH)], bufm)
        pltpu.sync_copy(bufm, outm.at[pl.ds(ro, ECH), pl.ds(lo, 128)])
        return carry

    lax.fori_loop(0, _ROWS_PER_TILE // ECH, drain, 0)
    plsc.subcore_barrier()

    pltpu.sync_copy(zm_hbm, bufm)

    def zcp2(t, carry):
        pltpu.sync_copy(bufm, acc_m.at[pl.ds(rs + t * ECH, ECH)])
        return carry

    lax.fori_loop(0, _ROWS_PER_TILE // ECH, zcp2, 0)
    plsc.subcore_barrier()

    def body2(k, carry):
        j = (s * 2 + c) + k * _NW

        @pl.when(j < NCH)
        def _():
            base = j * ECH
            pltpu.sync_copy(row_hbm.at[pl.ds(base, ECH)], idxv)
            pltpu.sync_copy(tr_hbm.at[pl.ds(base, ECH)], bufm)
            pltpu.sync_copy(bufm, acc_m.at[idxv], add=True)

        return carry

    lax.fori_loop(0, (NCH + _NW - 1) // _NW, body2, 0)
    plsc.subcore_barrier()

    def drain2(t, carry):
        ro = rs + t * ECH
        pltpu.sync_copy(acc_m.at[pl.ds(ro, ECH)], bufm)
        pltpu.sync_copy(bufm, outt.at[c, pl.ds(ro, ECH)])
        return carry

    lax.fori_loop(0, _ROWS_PER_TILE // ECH, drain2, 0)


@functools.cache
def _get_scatter_call():
  return functools.partial(
    pl.kernel,
    out_type=[
        jax.ShapeDtypeStruct((NPAD, HN), jnp.float32),
        jax.ShapeDtypeStruct((2, NPAD, 128), jnp.float32),
    ],
    mesh=plsc.VectorSubcoreMesh(core_axis_name="c", subcore_axis_name="s"),
    compiler_params=pltpu.CompilerParams(needs_layout_passes=False),
    scratch_types=[
        pltpu.VMEM((ECH,), jnp.int32),
        pltpu.VMEM((ECH, 128), jnp.float32),
        pltpu.VMEM_SHARED((NPAD, 128), jnp.float32),
    ],
  )(_scatter_body)


_BN = 1024
_BE = 512


def _b16(v):
    return v.astype(jnp.bfloat16).astype(jnp.float32)


def _emb_body(h, w, b, o):
    o[...] = h[...] @ w[...] + b[...]


def _ab_body(hh, wa, wb, oa, ob):
    v = hh[...]
    oa[...] = v @ wa[...]
    ob[...] = v @ wb[...]


def _edge_body(har, hbc, dif, ea, w1e, b1, wr, w2, b2, cw1, cb1, cw2,
               mo, tro):
    d = dif[...]
    radial = jnp.sum(d * d, axis=1, keepdims=True)
    pre = (har[...] + hbc[...] + _b16(radial) * _b16(wr[...])
           + ea[...] @ w1e[...] + b1[...])
    m1 = pre * jax.nn.sigmoid(pre)
    m2 = m1 @ w2[...] + b2[...]
    m = m2 * jax.nn.sigmoid(m2)
    u = m @ cw1[...] + cb1[...]
    u = u * jax.nn.sigmoid(u)
    phi = jnp.sum(_b16(u) * _b16(cw2[...]), axis=1, keepdims=True)
    lane = lax.broadcasted_iota(jnp.int32, d.shape, 1)
    t16 = d * phi + jnp.where(lane == 3, 1.0, 0.0)
    tro[...] = lax.pad(t16, jnp.float32(0.0), ((0, 0, 0), (0, 112, 0)))
    mo[...] = m


def _node_body2(hh, mg, ts, cp, vp,
                vw1, vb1, vw2, vb2, n1a, n1b, nb1, nw2, nb2,
                ho, co):
    hv = hh[...]
    p1 = hv @ vw1[...] + vb1[...]
    p1 = p1 * jax.nn.sigmoid(p1)
    pv = jnp.sum(_b16(p1) * _b16(vw2[...]), axis=1, keepdims=True) + vb2[...]
    t = ts[0] + ts[1]
    lane = lax.broadcasted_iota(jnp.int32, t.shape, 1)
    cnt = jnp.sum(jnp.where(lane == 3, t, 0.0), axis=1, keepdims=True)
    agg = jnp.where(lane < 3, t, 0.0) / jnp.maximum(cnt, 1.0)
    co[...] = cp[...] + agg * CW + pv * vp[...]
    o = hv @ n1a[...] + mg[...] @ n1b[...] + nb1[...]
    o = o * jax.nn.sigmoid(o)
    ho[...] = o @ nw2[...] + nb2[...]


def _full(shape):
    return pl.BlockSpec(shape, lambda i: (0,) * len(shape))


def _rows(shape):
    return pl.BlockSpec(shape, lambda i: (i,) + (0,) * (len(shape) - 1))


def _rows3(shape):
    return pl.BlockSpec(shape, lambda i: (0, i, 0))


def _emb_call(h, w, b):
    return pl.pallas_call(
        _emb_body,
        grid=(NPAD // _BN,),
        in_specs=[_rows((_BN, 128)), _full((128, HN)), _full((1, HN))],
        out_specs=_rows((_BN, HN)),
        out_shape=jax.ShapeDtypeStruct((NPAD, HN), jnp.float32),
    )(h, w, b)


def _ab_call(hh, wa, wb):
    return pl.pallas_call(
        _ab_body,
        grid=(NPAD // _BN,),
        in_specs=[_rows((_BN, HN)), _full((HN, HN)), _full((HN, HN))],
        out_specs=[_rows((_BN, HN)), _rows((_BN, HN))],
        out_shape=[jax.ShapeDtypeStruct((NPAD, HN), jnp.float32),
                   jax.ShapeDtypeStruct((NPAD, HN), jnp.float32)],
    )(hh, wa, wb)


def _edge_call(har, hbc, dif, ea, w1e, b1, wr, w2, b2, cw1, cb1, cw2):
    return pl.pallas_call(
        _edge_body,
        grid=(E // _BE,),
        in_specs=[
            _rows((_BE, HN)), _rows((_BE, HN)), _rows((_BE, 16)),
            _rows((_BE, 16)),
            _full((16, HN)), _full((1, HN)), _full((1, HN)),
            _full((HN, HN)), _full((1, HN)),
            _full((HN, HN)), _full((1, HN)), _full((1, HN)),
        ],
        out_specs=[_rows((_BE, HN)), _rows((_BE, 128))],
        out_shape=[jax.ShapeDtypeStruct((E, HN), jnp.float32),
                   jax.ShapeDtypeStruct((E, 128), jnp.float32)],
    )(har, hbc, dif, ea, w1e, b1, wr, w2, b2, cw1, cb1, cw2)


def _node_call(hh, mg, ts, cp, vp, lw):
    return pl.pallas_call(
        _node_body2,
        grid=(NPAD // _BN,),
        in_specs=[
            _rows((_BN, HN)), _rows((_BN, HN)), _rows3((2, _BN, 128)),
            _rows((_BN, 128)), _rows((_BN, 128)),
            _full((HN, HN)), _full((1, HN)), _full((1, HN)), _full((1, 1)),
            _full((HN, HN)), _full((HN, HN)),
            _full((1, HN)), _full((HN, HN)), _full((1, HN)),
        ],
        out_specs=[_rows((_BN, HN)), _rows((_BN, 128))],
        out_shape=[jax.ShapeDtypeStruct((NPAD, HN), jnp.float32),
                   jax.ShapeDtypeStruct((NPAD, 128), jnp.float32)],
    )(hh, mg, ts, cp, vp, *lw)


def kernel(h, x, edges, vel, edge_attr, params):
    row = edges[0]
    col = edges[1]

    hp = jnp.zeros((NPAD, 128), jnp.float32).at[:N].set(h)
    cp = jnp.zeros((NPAD, 128), jnp.float32).at[:N, :3].set(x[:, :, 0])
    vp = jnp.zeros((NPAD, 128), jnp.float32).at[:N, :3].set(vel)

    hh = _emb_call(hp, params['emb_W'], params['emb_b'][None, :])
    zm = jnp.zeros((ECH, 128), jnp.float32)

    for lp in params['layers']:
        w1 = lp['edge_W1']
        wa, wb = w1[:HN], w1[HN:2 * HN]
        wr = w1[2 * HN:2 * HN + 1]
        w1e = w1[2 * HN + 1:]
        ha, hb = _ab_call(hh, wa, wb)
        har, hbc, dif = _get_gather_call()(row, col, ha, hb, cp.T)
        m, tro = _edge_call(
            har, hbc, dif, edge_attr,
            w1e, lp['edge_b1'][None, :], wr,
            lp['edge_W2'], lp['edge_b2'][None, :],
            lp['coord_W1'], lp['coord_b1'][None, :], lp['coord_W2'].T)
        mg, ts = _get_scatter_call()(row, m, tro, zm)
        lw = (lp['vel_W1'], lp['vel_b1'][None, :], lp['vel_W2'].T,
              lp['vel_b2'][None, :],
              lp['node_W1'][:HN], lp['node_W1'][HN:],
              lp['node_b1'][None, :],
              lp['node_W2'], lp['node_b2'][None, :])
        hh, cp = _node_call(hh, mg, ts, cp, vp, lw)

    return cp[:N, :3]
